# baseline (speedup 1.0000x reference)
import functools

import jax
import jax.numpy as jnp
from jax import lax
from jax.experimental import pallas as pl
from jax.experimental.pallas import tpu as pltpu
from jax.experimental.pallas import tpu_sc as plsc

VOCAB = 100000
T = 36
EMB = 128
UNITS = 128
DENSE = 192
B = 1024

VT = 2048
NV = (VOCAB + VT - 1) // VT
VPAD = NV * VT


_SC_NC = 2
_SC_NS = 16
_SC_NW = _SC_NC * _SC_NS
_GB = T * B // _SC_NW
_GCH = 384
_GNCH = _GB // _GCH


def _sc_gather_kernel(table_hbm, idx_hbm, out_hbm, idx_v, rows0, rows1,
                      sem0, sem1):
    wid = lax.axis_index("s") * _SC_NC + lax.axis_index("c")
    base = wid * _GB
    pltpu.sync_copy(idx_hbm.at[pl.ds(base, _GB)], idx_v)
    bufs = (rows0, rows1)
    sems = (sem0, sem1)
    copies = [None] * _GNCH
    copies[0] = pltpu.async_copy(
        table_hbm.at[idx_v.at[pl.ds(0, _GCH)]], bufs[0], sems[0])
    for c in range(_GNCH):
        if c + 1 < _GNCH:
            copies[c + 1] = pltpu.async_copy(
                table_hbm.at[idx_v.at[pl.ds((c + 1) * _GCH, _GCH)]],
                bufs[(c + 1) % 2], sems[(c + 1) % 2])
        copies[c].wait()
        pltpu.sync_copy(bufs[c % 2],
                        out_hbm.at[pl.ds(base + c * _GCH, _GCH)])


def _sc_gather(emb_table, flat_idx):
    k = functools.partial(
        pl.kernel,
        out_type=jax.ShapeDtypeStruct((T * B, EMB), jnp.float32),
        mesh=plsc.VectorSubcoreMesh(core_axis_name="c", subcore_axis_name="s"),
        scratch_types=[
            pltpu.VMEM((_GB,), jnp.int32),
            pltpu.VMEM((_GCH, EMB), jnp.float32),
            pltpu.VMEM((_GCH, EMB), jnp.float32),
            pltpu.SemaphoreType.DMA,
            pltpu.SemaphoreType.DMA,
        ],
    )(_sc_gather_kernel)
    return k(emb_table, flat_idx)


def _lstm_step_kernel(xf_ref, xb_ref, Wfk_ref, Wfr_ref, bf_ref,
                      Wbk_ref, Wbr_ref, bb_ref, Wd_ref, bd_ref,
                      d_out_ref, hf_ref, cf_ref, hb_ref, cb_ref):
    t = pl.program_id(0)

    @pl.when(t == 0)
    def _init():
        hf_ref[...] = jnp.zeros_like(hf_ref)
        cf_ref[...] = jnp.zeros_like(cf_ref)
        hb_ref[...] = jnp.zeros_like(hb_ref)
        cb_ref[...] = jnp.zeros_like(cb_ref)

    def step(x16, h_ref, c_ref, Wk_ref, Wr_ref, b_ref):
        h16 = h_ref[...].astype(jnp.bfloat16)
        z = (jnp.dot(x16, Wk_ref[...], preferred_element_type=jnp.float32)
             + jnp.dot(h16, Wr_ref[...], preferred_element_type=jnp.float32)
             + b_ref[...])
        i = jax.nn.sigmoid(z[:, 0 * UNITS:1 * UNITS])
        f = jax.nn.sigmoid(z[:, 1 * UNITS:2 * UNITS])
        g = jnp.tanh(z[:, 2 * UNITS:3 * UNITS])
        o = jax.nn.sigmoid(z[:, 3 * UNITS:4 * UNITS])
        c_new = f * c_ref[...] + i * g
        h_new = o * jnp.tanh(c_new)
        h_ref[...] = h_new
        c_ref[...] = c_new
        return h_new

    hf = step(xf_ref[0].astype(jnp.bfloat16), hf_ref, cf_ref,
              Wfk_ref, Wfr_ref, bf_ref)
    hb = step(xb_ref[0].astype(jnp.bfloat16), hb_ref, cb_ref,
              Wbk_ref, Wbr_ref, bb_ref)

    @pl.when(t == T - 1)
    def _emit():
        d_pre = (jnp.dot(hf.astype(jnp.bfloat16), Wd_ref[0:UNITS, :],
                         preferred_element_type=jnp.float32)
                 + jnp.dot(hb.astype(jnp.bfloat16), Wd_ref[UNITS:2 * UNITS, :],
                           preferred_element_type=jnp.float32)
                 + bd_ref[...])
        d = jnp.where(d_pre > 0, d_pre, 0.1 * d_pre)
        d_out_ref[...] = d.astype(jnp.bfloat16)


def _run_lstm(x_tm, Wf_k, Wf_r, bf, Wb_k, Wb_r, bb, Wd, bd):
    full = lambda shape: pl.BlockSpec(shape, lambda t: tuple(0 for _ in shape))
    return pl.pallas_call(
        _lstm_step_kernel,
        grid=(T,),
        in_specs=[
            pl.BlockSpec((1, B, EMB), lambda t: (t, 0, 0)),
            pl.BlockSpec((1, B, EMB), lambda t: (T - 1 - t, 0, 0)),
            full((EMB, 4 * UNITS)),
            full((UNITS, 4 * UNITS)),
            full((1, 4 * UNITS)),
            full((EMB, 4 * UNITS)),
            full((UNITS, 4 * UNITS)),
            full((1, 4 * UNITS)),
            full((2 * UNITS, DENSE)),
            full((1, DENSE)),
        ],
        out_specs=pl.BlockSpec((B, DENSE), lambda t: (0, 0)),
        out_shape=jax.ShapeDtypeStruct((B, DENSE), jnp.bfloat16),
        scratch_shapes=[
            pltpu.VMEM((B, UNITS), jnp.float32),
            pltpu.VMEM((B, UNITS), jnp.float32),
            pltpu.VMEM((B, UNITS), jnp.float32),
            pltpu.VMEM((B, UNITS), jnp.float32),
        ],
    )(x_tm, x_tm, Wf_k, Wf_r, bf, Wb_k, Wb_r, bb, Wd, bd)


def _logits_tile(wo_ref, bo_ref, ones_ref, dT_ref):
    w16 = wo_ref[...].astype(jnp.bfloat16)
    tn = (((0,), (0,)), ((), ()))
    l = jax.lax.dot_general(w16, dT_ref[...], dimension_numbers=tn,
                            preferred_element_type=jnp.float32)
    b = jax.lax.dot_general(bo_ref[...].astype(jnp.bfloat16), ones_ref[...],
                            dimension_numbers=tn,
                            preferred_element_type=jnp.float32)
    return l + b


def _head_pass1_kernel(wo_ref, bo_ref, ones_ref, dT_ref, m_out_ref, s_out_ref,
                       m_ref, s_ref):
    j = pl.program_id(0)
    l = _logits_tile(wo_ref, bo_ref, ones_ref, dT_ref)
    row = jax.lax.broadcasted_iota(jnp.int32, (VT, 1), 0)
    l = jnp.where(row < VOCAB - j * VT, l, -1e30)
    m_tile = jnp.max(l, axis=0, keepdims=True)

    def _sumexp(x, m):
        e = jnp.exp((x - m).astype(jnp.bfloat16))
        return jnp.sum(e.astype(jnp.float32), axis=0, keepdims=True)

    @pl.when(j == 0)
    def _first():
        m_ref[...] = m_tile
        s_ref[...] = _sumexp(l, m_tile)

    @pl.when(j > 0)
    def _rest():
        m_old = m_ref[...]
        m_new = jnp.maximum(m_old, m_tile)
        s_ref[...] = (s_ref[...] * jnp.exp(m_old - m_new)
                      + _sumexp(l, m_new))
        m_ref[...] = m_new

    @pl.when(j == NV - 1)
    def _emit():
        m_out_ref[...] = m_ref[...]
        s_out_ref[...] = 1.0 / s_ref[...]


def _head_pass2_kernel(wo_ref, bo_ref, ones_ref, dT_ref, m_ref, sinv_ref,
                       out_ref):
    l = _logits_tile(wo_ref, bo_ref, ones_ref, dT_ref)
    out_ref[...] = jnp.exp(l - m_ref[...]) * sinv_ref[...]


def _run_head(dT, Wo, bo_row):
    wt_spec = pl.BlockSpec((DENSE, VT), lambda j: (0, j))
    bo_spec = pl.BlockSpec((1, VT), lambda j: (0, j))
    dT_spec = pl.BlockSpec((DENSE, B), lambda j: (0, 0))
    row_spec = pl.BlockSpec((1, B), lambda j: (0, 0))
    ones_row = jnp.ones((1, B), jnp.bfloat16)

    m, sinv = pl.pallas_call(
        _head_pass1_kernel,
        grid=(NV,),
        in_specs=[wt_spec, bo_spec, row_spec, dT_spec],
        out_specs=[row_spec, row_spec],
        out_shape=[jax.ShapeDtypeStruct((1, B), jnp.float32),
                   jax.ShapeDtypeStruct((1, B), jnp.float32)],
        scratch_shapes=[pltpu.VMEM((1, B), jnp.float32),
                        pltpu.VMEM((1, B), jnp.float32)],
    )(Wo, bo_row, ones_row, dT)

    out_t = pl.pallas_call(
        _head_pass2_kernel,
        grid=(NV,),
        in_specs=[wt_spec, bo_spec, row_spec, dT_spec, row_spec, row_spec],
        out_specs=pl.BlockSpec((VT, B), lambda j: (j, 0)),
        out_shape=jax.ShapeDtypeStruct((VOCAB, B), jnp.float32),
    )(Wo, bo_row, ones_row, dT, m, sinv)
    return out_t


def kernel(inputs, training, emb_table, Wf_k, Wf_r, bf, Wb_k, Wb_r, bb,
           Wd, bd, Wo, bo):
    del training
    flat_idx = inputs.T.reshape(-1)
    x_tm = _sc_gather(emb_table, flat_idx).reshape(T, B, EMB)

    b16 = lambda w: w.astype(jnp.bfloat16)
    d = _run_lstm(x_tm, b16(Wf_k), b16(Wf_r), bf.reshape(1, -1),
                  b16(Wb_k), b16(Wb_r), bb.reshape(1, -1),
                  b16(Wd), bd.reshape(1, -1))
    dT = d.T

    out_t = _run_head(dT, Wo, bo.reshape(1, -1))
    return out_t.T


# speedup vs baseline: 1.0773x; 1.0773x over previous
import functools

import jax
import jax.numpy as jnp
from jax import lax
from jax.experimental import pallas as pl
from jax.experimental.pallas import tpu as pltpu
from jax.experimental.pallas import tpu_sc as plsc

VOCAB = 100000
T = 36
EMB = 128
UNITS = 128
DENSE = 192
B = 1024

VT = 2048
NV = (VOCAB + VT - 1) // VT
VPAD = NV * VT


_SC_NC = 2
_SC_NS = 16
_SC_NW = _SC_NC * _SC_NS
_GB = T * B // _SC_NW
_GCH = 384
_GNCH = _GB // _GCH


def _sc_gather_kernel(table_hbm, idx_hbm, out_hbm, idx_v, rows0, rows1,
                      sem0, sem1):
    wid = lax.axis_index("s") * _SC_NC + lax.axis_index("c")
    base = wid * _GB
    pltpu.sync_copy(idx_hbm.at[pl.ds(base, _GB)], idx_v)
    bufs = (rows0, rows1)
    sems = (sem0, sem1)
    copies = [None] * _GNCH
    copies[0] = pltpu.async_copy(
        table_hbm.at[idx_v.at[pl.ds(0, _GCH)]], bufs[0], sems[0])
    for c in range(_GNCH):
        if c + 1 < _GNCH:
            copies[c + 1] = pltpu.async_copy(
                table_hbm.at[idx_v.at[pl.ds((c + 1) * _GCH, _GCH)]],
                bufs[(c + 1) % 2], sems[(c + 1) % 2])
        copies[c].wait()
        pltpu.sync_copy(bufs[c % 2],
                        out_hbm.at[pl.ds(base + c * _GCH, _GCH)])


def _sc_gather(emb_table, flat_idx):
    k = functools.partial(
        pl.kernel,
        out_type=jax.ShapeDtypeStruct((T * B, EMB), jnp.float32),
        mesh=plsc.VectorSubcoreMesh(core_axis_name="c", subcore_axis_name="s"),
        scratch_types=[
            pltpu.VMEM((_GB,), jnp.int32),
            pltpu.VMEM((_GCH, EMB), jnp.float32),
            pltpu.VMEM((_GCH, EMB), jnp.float32),
            pltpu.SemaphoreType.DMA,
            pltpu.SemaphoreType.DMA,
        ],
    )(_sc_gather_kernel)
    return k(emb_table, flat_idx)


def _lstm_step_kernel(xf_ref, xb_ref, Wfk_ref, Wfr_ref, bf_ref,
                      Wbk_ref, Wbr_ref, bb_ref, Wd_ref, bd_ref,
                      d_out_ref, hf_ref, cf_ref, hb_ref, cb_ref):
    t = pl.program_id(0)

    @pl.when(t == 0)
    def _init():
        hf_ref[...] = jnp.zeros_like(hf_ref)
        cf_ref[...] = jnp.zeros_like(cf_ref)
        hb_ref[...] = jnp.zeros_like(hb_ref)
        cb_ref[...] = jnp.zeros_like(cb_ref)

    def step(x16, h_ref, c_ref, Wk_ref, Wr_ref, b_ref):
        h16 = h_ref[...].astype(jnp.bfloat16)
        z = (jnp.dot(x16, Wk_ref[...], preferred_element_type=jnp.float32)
             + jnp.dot(h16, Wr_ref[...], preferred_element_type=jnp.float32)
             + b_ref[...])
        i = jax.nn.sigmoid(z[:, 0 * UNITS:1 * UNITS])
        f = jax.nn.sigmoid(z[:, 1 * UNITS:2 * UNITS])
        g = jnp.tanh(z[:, 2 * UNITS:3 * UNITS])
        o = jax.nn.sigmoid(z[:, 3 * UNITS:4 * UNITS])
        c_new = f * c_ref[...] + i * g
        h_new = o * jnp.tanh(c_new)
        h_ref[...] = h_new
        c_ref[...] = c_new
        return h_new

    hf = step(xf_ref[0].astype(jnp.bfloat16), hf_ref, cf_ref,
              Wfk_ref, Wfr_ref, bf_ref)
    hb = step(xb_ref[0].astype(jnp.bfloat16), hb_ref, cb_ref,
              Wbk_ref, Wbr_ref, bb_ref)

    @pl.when(t == T - 1)
    def _emit():
        d_pre = (jnp.dot(hf.astype(jnp.bfloat16), Wd_ref[0:UNITS, :],
                         preferred_element_type=jnp.float32)
                 + jnp.dot(hb.astype(jnp.bfloat16), Wd_ref[UNITS:2 * UNITS, :],
                           preferred_element_type=jnp.float32)
                 + bd_ref[...])
        d = jnp.where(d_pre > 0, d_pre, 0.1 * d_pre)
        d_out_ref[...] = d.astype(jnp.bfloat16)


def _run_lstm(x_tm, Wf_k, Wf_r, bf, Wb_k, Wb_r, bb, Wd, bd):
    full = lambda shape: pl.BlockSpec(shape, lambda t: tuple(0 for _ in shape))
    return pl.pallas_call(
        _lstm_step_kernel,
        grid=(T,),
        in_specs=[
            pl.BlockSpec((1, B, EMB), lambda t: (t, 0, 0)),
            pl.BlockSpec((1, B, EMB), lambda t: (T - 1 - t, 0, 0)),
            full((EMB, 4 * UNITS)),
            full((UNITS, 4 * UNITS)),
            full((1, 4 * UNITS)),
            full((EMB, 4 * UNITS)),
            full((UNITS, 4 * UNITS)),
            full((1, 4 * UNITS)),
            full((2 * UNITS, DENSE)),
            full((1, DENSE)),
        ],
        out_specs=pl.BlockSpec((B, DENSE), lambda t: (0, 0)),
        out_shape=jax.ShapeDtypeStruct((B, DENSE), jnp.bfloat16),
        scratch_shapes=[
            pltpu.VMEM((B, UNITS), jnp.float32),
            pltpu.VMEM((B, UNITS), jnp.float32),
            pltpu.VMEM((B, UNITS), jnp.float32),
            pltpu.VMEM((B, UNITS), jnp.float32),
        ],
    )(x_tm, x_tm, Wf_k, Wf_r, bf, Wb_k, Wb_r, bb, Wd, bd)


def _logits_tile(wo_ref, bo_ref, ones_ref, dT_ref):
    w16 = wo_ref[...].astype(jnp.bfloat16)
    tn = (((0,), (0,)), ((), ()))
    l = jax.lax.dot_general(w16, dT_ref[...], dimension_numbers=tn,
                            preferred_element_type=jnp.float32)
    b = jax.lax.dot_general(bo_ref[...].astype(jnp.bfloat16), ones_ref[...],
                            dimension_numbers=tn,
                            preferred_element_type=jnp.float32)
    return l + b


def _head_pass1_kernel(wo_ref, bo_ref, ones_ref, dT_ref, m_out_ref, s_out_ref,
                       m_ref, s_ref):
    j = pl.program_id(0)
    l = _logits_tile(wo_ref, bo_ref, ones_ref, dT_ref)
    row = jax.lax.broadcasted_iota(jnp.int32, (VT, 1), 0)
    l = jnp.where(row < VOCAB - j * VT, l, -1e30)
    m_tile = jnp.max(l, axis=0, keepdims=True)

    def _sumexp(x, m):
        return jnp.sum(jnp.exp(x - m), axis=0, keepdims=True)

    @pl.when(j == 0)
    def _first():
        m_ref[...] = m_tile
        s_ref[...] = _sumexp(l, m_tile)

    @pl.when(j > 0)
    def _rest():
        m_old = m_ref[...]
        m_new = jnp.maximum(m_old, m_tile)
        s_ref[...] = (s_ref[...] * jnp.exp(m_old - m_new)
                      + _sumexp(l, m_new))
        m_ref[...] = m_new

    @pl.when(j == NV - 1)
    def _emit():
        m_out_ref[...] = m_ref[...]
        s_out_ref[...] = 1.0 / s_ref[...]


def _head_pass2_kernel(wo_ref, bo_ref, ones_ref, dT_ref, m_ref, sinv_ref,
                       out_ref):
    l = _logits_tile(wo_ref, bo_ref, ones_ref, dT_ref)
    out_ref[...] = jnp.exp(l - m_ref[...]) * sinv_ref[...]


def _run_head(dT, Wo, bo_row):
    wt_spec = pl.BlockSpec((DENSE, VT), lambda j: (0, j))
    bo_spec = pl.BlockSpec((1, VT), lambda j: (0, j))
    dT_spec = pl.BlockSpec((DENSE, B), lambda j: (0, 0))
    row_spec = pl.BlockSpec((1, B), lambda j: (0, 0))
    ones_row = jnp.ones((1, B), jnp.bfloat16)

    m, sinv = pl.pallas_call(
        _head_pass1_kernel,
        grid=(NV,),
        in_specs=[wt_spec, bo_spec, row_spec, dT_spec],
        out_specs=[row_spec, row_spec],
        out_shape=[jax.ShapeDtypeStruct((1, B), jnp.float32),
                   jax.ShapeDtypeStruct((1, B), jnp.float32)],
        scratch_shapes=[pltpu.VMEM((1, B), jnp.float32),
                        pltpu.VMEM((1, B), jnp.float32)],
    )(Wo, bo_row, ones_row, dT)

    out_t = pl.pallas_call(
        _head_pass2_kernel,
        grid=(NV,),
        in_specs=[wt_spec, bo_spec, row_spec, dT_spec, row_spec, row_spec],
        out_specs=pl.BlockSpec((VT, B), lambda j: (j, 0)),
        out_shape=jax.ShapeDtypeStruct((VOCAB, B), jnp.float32),
    )(Wo, bo_row, ones_row, dT, m, sinv)
    return out_t


def kernel(inputs, training, emb_table, Wf_k, Wf_r, bf, Wb_k, Wb_r, bb,
           Wd, bd, Wo, bo):
    del training
    flat_idx = inputs.T.reshape(-1)
    x_tm = _sc_gather(emb_table, flat_idx).reshape(T, B, EMB)

    b16 = lambda w: w.astype(jnp.bfloat16)
    d = _run_lstm(x_tm, b16(Wf_k), b16(Wf_r), bf.reshape(1, -1),
                  b16(Wb_k), b16(Wb_r), bb.reshape(1, -1),
                  b16(Wd), bd.reshape(1, -1))
    dT = d.T

    out_t = _run_head(dT, Wo, bo.reshape(1, -1))
    return out_t.T


# speedup vs baseline: 1.0972x; 1.0185x over previous
import functools

import jax
import jax.numpy as jnp
from jax import lax
from jax.experimental import pallas as pl
from jax.experimental.pallas import tpu as pltpu
from jax.experimental.pallas import tpu_sc as plsc

VOCAB = 100000
T = 36
EMB = 128
UNITS = 128
DENSE = 192
B = 1024

VT = 2048
NV = (VOCAB + VT - 1) // VT
VPAD = NV * VT


_SC_NC = 2
_SC_NS = 16
_SC_NW = _SC_NC * _SC_NS
_GB = T * B // _SC_NW
_GCH = 384
_GNCH = _GB // _GCH


def _sc_gather_kernel(table_hbm, idx_hbm, out_hbm, idx_v, rows0, rows1,
                      sem0, sem1):
    wid = lax.axis_index("s") * _SC_NC + lax.axis_index("c")
    base = wid * _GB
    pltpu.sync_copy(idx_hbm.at[pl.ds(base, _GB)], idx_v)
    bufs = (rows0, rows1)
    sems = (sem0, sem1)
    copies = [None] * _GNCH
    copies[0] = pltpu.async_copy(
        table_hbm.at[idx_v.at[pl.ds(0, _GCH)]], bufs[0], sems[0])
    for c in range(_GNCH):
        if c + 1 < _GNCH:
            copies[c + 1] = pltpu.async_copy(
                table_hbm.at[idx_v.at[pl.ds((c + 1) * _GCH, _GCH)]],
                bufs[(c + 1) % 2], sems[(c + 1) % 2])
        copies[c].wait()
        pltpu.sync_copy(bufs[c % 2],
                        out_hbm.at[pl.ds(base + c * _GCH, _GCH)])


def _sc_gather(emb_table, flat_idx):
    k = functools.partial(
        pl.kernel,
        out_type=jax.ShapeDtypeStruct((T * B, EMB), jnp.float32),
        mesh=plsc.VectorSubcoreMesh(core_axis_name="c", subcore_axis_name="s"),
        scratch_types=[
            pltpu.VMEM((_GB,), jnp.int32),
            pltpu.VMEM((_GCH, EMB), jnp.float32),
            pltpu.VMEM((_GCH, EMB), jnp.float32),
            pltpu.SemaphoreType.DMA,
            pltpu.SemaphoreType.DMA,
        ],
    )(_sc_gather_kernel)
    return k(emb_table, flat_idx)


def _lstm_step_kernel(xf_ref, xb_ref, Wfk_ref, Wfr_ref, bf_ref,
                      Wbk_ref, Wbr_ref, bb_ref, Wd_ref, bd_ref,
                      d_out_ref, hf_ref, cf_ref, hb_ref, cb_ref):
    t = pl.program_id(0)

    @pl.when(t == 0)
    def _init():
        hf_ref[...] = jnp.zeros_like(hf_ref)
        cf_ref[...] = jnp.zeros_like(cf_ref)
        hb_ref[...] = jnp.zeros_like(hb_ref)
        cb_ref[...] = jnp.zeros_like(cb_ref)

    def step(x16, h_ref, c_ref, Wk_ref, Wr_ref, b_ref):
        h16 = h_ref[...].astype(jnp.bfloat16)
        z = (jnp.dot(x16, Wk_ref[...], preferred_element_type=jnp.float32)
             + jnp.dot(h16, Wr_ref[...], preferred_element_type=jnp.float32)
             + b_ref[...])
        i = jax.nn.sigmoid(z[:, 0 * UNITS:1 * UNITS])
        f = jax.nn.sigmoid(z[:, 1 * UNITS:2 * UNITS])
        g = jnp.tanh(z[:, 2 * UNITS:3 * UNITS])
        o = jax.nn.sigmoid(z[:, 3 * UNITS:4 * UNITS])
        c_new = f * c_ref[...] + i * g
        h_new = o * jnp.tanh(c_new)
        h_ref[...] = h_new
        c_ref[...] = c_new
        return h_new

    step(xf_ref[0].astype(jnp.bfloat16), hf_ref, cf_ref,
         Wfk_ref, Wfr_ref, bf_ref)
    step(xb_ref[1].astype(jnp.bfloat16), hb_ref, cb_ref,
         Wbk_ref, Wbr_ref, bb_ref)
    hf = step(xf_ref[1].astype(jnp.bfloat16), hf_ref, cf_ref,
              Wfk_ref, Wfr_ref, bf_ref)
    hb = step(xb_ref[0].astype(jnp.bfloat16), hb_ref, cb_ref,
              Wbk_ref, Wbr_ref, bb_ref)

    @pl.when(t == T // 2 - 1)
    def _emit():
        d_pre = (jnp.dot(hf.astype(jnp.bfloat16), Wd_ref[0:UNITS, :],
                         preferred_element_type=jnp.float32)
                 + jnp.dot(hb.astype(jnp.bfloat16), Wd_ref[UNITS:2 * UNITS, :],
                           preferred_element_type=jnp.float32)
                 + bd_ref[...])
        d = jnp.where(d_pre > 0, d_pre, 0.1 * d_pre)
        d_out_ref[...] = d.astype(jnp.bfloat16)


def _run_lstm(x_tm, Wf_k, Wf_r, bf, Wb_k, Wb_r, bb, Wd, bd):
    full = lambda shape: pl.BlockSpec(shape, lambda t: tuple(0 for _ in shape))
    return pl.pallas_call(
        _lstm_step_kernel,
        grid=(T // 2,),
        in_specs=[
            pl.BlockSpec((2, B, EMB), lambda t: (t, 0, 0)),
            pl.BlockSpec((2, B, EMB), lambda t: (T // 2 - 1 - t, 0, 0)),
            full((EMB, 4 * UNITS)),
            full((UNITS, 4 * UNITS)),
            full((1, 4 * UNITS)),
            full((EMB, 4 * UNITS)),
            full((UNITS, 4 * UNITS)),
            full((1, 4 * UNITS)),
            full((2 * UNITS, DENSE)),
            full((1, DENSE)),
        ],
        out_specs=pl.BlockSpec((B, DENSE), lambda t: (0, 0)),
        out_shape=jax.ShapeDtypeStruct((B, DENSE), jnp.bfloat16),
        scratch_shapes=[
            pltpu.VMEM((B, UNITS), jnp.float32),
            pltpu.VMEM((B, UNITS), jnp.float32),
            pltpu.VMEM((B, UNITS), jnp.float32),
            pltpu.VMEM((B, UNITS), jnp.float32),
        ],
    )(x_tm, x_tm, Wf_k, Wf_r, bf, Wb_k, Wb_r, bb, Wd, bd)


def _logits_tile(wo_ref, bo_ref, ones_ref, dT_ref):
    w16 = wo_ref[...].astype(jnp.bfloat16)
    tn = (((0,), (0,)), ((), ()))
    l = jax.lax.dot_general(w16, dT_ref[...], dimension_numbers=tn,
                            preferred_element_type=jnp.float32)
    b = jax.lax.dot_general(bo_ref[...].astype(jnp.bfloat16), ones_ref[...],
                            dimension_numbers=tn,
                            preferred_element_type=jnp.float32)
    return l + b


def _head_pass1_kernel(wo_ref, bo_ref, ones_ref, dT_ref, m_out_ref, s_out_ref,
                       m_ref, s_ref):
    j = pl.program_id(0)
    l = _logits_tile(wo_ref, bo_ref, ones_ref, dT_ref)
    row = jax.lax.broadcasted_iota(jnp.int32, (VT, 1), 0)
    l = jnp.where(row < VOCAB - j * VT, l, -1e30)
    m_tile = jnp.max(l, axis=0, keepdims=True)

    def _sumexp(x, m):
        return jnp.sum(jnp.exp(x - m), axis=0, keepdims=True)

    @pl.when(j == 0)
    def _first():
        m_ref[...] = m_tile
        s_ref[...] = _sumexp(l, m_tile)

    @pl.when(j > 0)
    def _rest():
        m_old = m_ref[...]
        m_new = jnp.maximum(m_old, m_tile)
        s_ref[...] = (s_ref[...] * jnp.exp(m_old - m_new)
                      + _sumexp(l, m_new))
        m_ref[...] = m_new

    @pl.when(j == NV - 1)
    def _emit():
        m_out_ref[...] = m_ref[...]
        s_out_ref[...] = 1.0 / s_ref[...]


def _head_pass2_kernel(wo_ref, bo_ref, ones_ref, dT_ref, m_ref, sinv_ref,
                       out_ref):
    l = _logits_tile(wo_ref, bo_ref, ones_ref, dT_ref)
    out_ref[...] = jnp.exp(l - m_ref[...]) * sinv_ref[...]


def _run_head(dT, Wo, bo_row):
    wt_spec = pl.BlockSpec((DENSE, VT), lambda j: (0, j))
    bo_spec = pl.BlockSpec((1, VT), lambda j: (0, j))
    dT_spec = pl.BlockSpec((DENSE, B), lambda j: (0, 0))
    row_spec = pl.BlockSpec((1, B), lambda j: (0, 0))
    ones_row = jnp.ones((1, B), jnp.bfloat16)

    m, sinv = pl.pallas_call(
        _head_pass1_kernel,
        grid=(NV,),
        in_specs=[wt_spec, bo_spec, row_spec, dT_spec],
        out_specs=[row_spec, row_spec],
        out_shape=[jax.ShapeDtypeStruct((1, B), jnp.float32),
                   jax.ShapeDtypeStruct((1, B), jnp.float32)],
        scratch_shapes=[pltpu.VMEM((1, B), jnp.float32),
                        pltpu.VMEM((1, B), jnp.float32)],
    )(Wo, bo_row, ones_row, dT)

    out_t = pl.pallas_call(
        _head_pass2_kernel,
        grid=(NV,),
        in_specs=[wt_spec, bo_spec, row_spec, dT_spec, row_spec, row_spec],
        out_specs=pl.BlockSpec((VT, B), lambda j: (j, 0)),
        out_shape=jax.ShapeDtypeStruct((VOCAB, B), jnp.float32),
    )(Wo, bo_row, ones_row, dT, m, sinv)
    return out_t


def kernel(inputs, training, emb_table, Wf_k, Wf_r, bf, Wb_k, Wb_r, bb,
           Wd, bd, Wo, bo):
    del training
    flat_idx = inputs.T.reshape(-1)
    x_tm = _sc_gather(emb_table, flat_idx).reshape(T, B, EMB)

    b16 = lambda w: w.astype(jnp.bfloat16)
    d = _run_lstm(x_tm, b16(Wf_k), b16(Wf_r), bf.reshape(1, -1),
                  b16(Wb_k), b16(Wb_r), bb.reshape(1, -1),
                  b16(Wd), bd.reshape(1, -1))
    dT = d.T

    out_t = _run_head(dT, Wo, bo.reshape(1, -1))
    return out_t.T


# speedup vs baseline: 1.1018x; 1.0041x over previous
import functools

import jax
import jax.numpy as jnp
from jax import lax
from jax.experimental import pallas as pl
from jax.experimental.pallas import tpu as pltpu
from jax.experimental.pallas import tpu_sc as plsc

VOCAB = 100000
T = 36
EMB = 128
UNITS = 128
DENSE = 192
B = 1024

VT = 2048
NV = (VOCAB + VT - 1) // VT
VPAD = NV * VT


_SC_NC = 2
_SC_NS = 16
_SC_NW = _SC_NC * _SC_NS
_GB = T * B // _SC_NW
_GCH = 384
_GNCH = _GB // _GCH


def _sc_gather_kernel(table_hbm, idx_hbm, out_hbm, idx_v, rows0, rows1,
                      sem0, sem1):
    wid = lax.axis_index("s") * _SC_NC + lax.axis_index("c")
    base = wid * _GB
    pltpu.sync_copy(idx_hbm.at[pl.ds(base, _GB)], idx_v)
    bufs = (rows0, rows1)
    sems = (sem0, sem1)
    copies = [None] * _GNCH
    copies[0] = pltpu.async_copy(
        table_hbm.at[idx_v.at[pl.ds(0, _GCH)]], bufs[0], sems[0])
    for c in range(_GNCH):
        if c + 1 < _GNCH:
            copies[c + 1] = pltpu.async_copy(
                table_hbm.at[idx_v.at[pl.ds((c + 1) * _GCH, _GCH)]],
                bufs[(c + 1) % 2], sems[(c + 1) % 2])
        copies[c].wait()
        pltpu.sync_copy(bufs[c % 2],
                        out_hbm.at[pl.ds(base + c * _GCH, _GCH)])


def _sc_gather(emb_table, flat_idx):
    k = functools.partial(
        pl.kernel,
        out_type=jax.ShapeDtypeStruct((T * B, EMB), jnp.float32),
        mesh=plsc.VectorSubcoreMesh(core_axis_name="c", subcore_axis_name="s"),
        scratch_types=[
            pltpu.VMEM((_GB,), jnp.int32),
            pltpu.VMEM((_GCH, EMB), jnp.float32),
            pltpu.VMEM((_GCH, EMB), jnp.float32),
            pltpu.SemaphoreType.DMA,
            pltpu.SemaphoreType.DMA,
        ],
    )(_sc_gather_kernel)
    return k(emb_table, flat_idx)


_TSTEP = 4


def _lstm_step_kernel(xf_ref, xb_ref, Wfk_ref, Wfr_ref, bf_ref,
                      Wbk_ref, Wbr_ref, bb_ref, Wd_ref, bd_ref,
                      d_out_ref, hf_ref, cf_ref, hb_ref, cb_ref):
    t = pl.program_id(0)

    @pl.when(t == 0)
    def _init():
        hf_ref[...] = jnp.zeros_like(hf_ref)
        cf_ref[...] = jnp.zeros_like(cf_ref)
        hb_ref[...] = jnp.zeros_like(hb_ref)
        cb_ref[...] = jnp.zeros_like(cb_ref)

    def step(x16, h_ref, c_ref, Wk_ref, Wr_ref, b_ref):
        h16 = h_ref[...].astype(jnp.bfloat16)
        z = (jnp.dot(x16, Wk_ref[...], preferred_element_type=jnp.float32)
             + jnp.dot(h16, Wr_ref[...], preferred_element_type=jnp.float32)
             + b_ref[...])
        i = jax.nn.sigmoid(z[:, 0 * UNITS:1 * UNITS])
        f = jax.nn.sigmoid(z[:, 1 * UNITS:2 * UNITS])
        g = jnp.tanh(z[:, 2 * UNITS:3 * UNITS])
        o = jax.nn.sigmoid(z[:, 3 * UNITS:4 * UNITS])
        c_new = f * c_ref[...] + i * g
        h_new = o * jnp.tanh(c_new)
        h_ref[...] = h_new
        c_ref[...] = c_new
        return h_new

    for u in range(_TSTEP):
        hf = step(xf_ref[u].astype(jnp.bfloat16), hf_ref, cf_ref,
                  Wfk_ref, Wfr_ref, bf_ref)
        hb = step(xb_ref[_TSTEP - 1 - u].astype(jnp.bfloat16), hb_ref, cb_ref,
                  Wbk_ref, Wbr_ref, bb_ref)

    @pl.when(t == T // _TSTEP - 1)
    def _emit():
        d_pre = (jnp.dot(hf.astype(jnp.bfloat16), Wd_ref[0:UNITS, :],
                         preferred_element_type=jnp.float32)
                 + jnp.dot(hb.astype(jnp.bfloat16), Wd_ref[UNITS:2 * UNITS, :],
                           preferred_element_type=jnp.float32)
                 + bd_ref[...])
        d = jnp.where(d_pre > 0, d_pre, 0.1 * d_pre)
        d_out_ref[...] = d.astype(jnp.bfloat16)


def _run_lstm(x_tm, Wf_k, Wf_r, bf, Wb_k, Wb_r, bb, Wd, bd):
    full = lambda shape: pl.BlockSpec(shape, lambda t: tuple(0 for _ in shape))
    return pl.pallas_call(
        _lstm_step_kernel,
        grid=(T // _TSTEP,),
        in_specs=[
            pl.BlockSpec((_TSTEP, B, EMB), lambda t: (t, 0, 0)),
            pl.BlockSpec((_TSTEP, B, EMB),
                         lambda t: (T // _TSTEP - 1 - t, 0, 0)),
            full((EMB, 4 * UNITS)),
            full((UNITS, 4 * UNITS)),
            full((1, 4 * UNITS)),
            full((EMB, 4 * UNITS)),
            full((UNITS, 4 * UNITS)),
            full((1, 4 * UNITS)),
            full((2 * UNITS, DENSE)),
            full((1, DENSE)),
        ],
        out_specs=pl.BlockSpec((B, DENSE), lambda t: (0, 0)),
        out_shape=jax.ShapeDtypeStruct((B, DENSE), jnp.bfloat16),
        scratch_shapes=[
            pltpu.VMEM((B, UNITS), jnp.float32),
            pltpu.VMEM((B, UNITS), jnp.float32),
            pltpu.VMEM((B, UNITS), jnp.float32),
            pltpu.VMEM((B, UNITS), jnp.float32),
        ],
    )(x_tm, x_tm, Wf_k, Wf_r, bf, Wb_k, Wb_r, bb, Wd, bd)


def _logits_tile(wo_ref, bo_ref, ones_ref, dT_ref):
    w16 = wo_ref[...].astype(jnp.bfloat16)
    tn = (((0,), (0,)), ((), ()))
    l = jax.lax.dot_general(w16, dT_ref[...], dimension_numbers=tn,
                            preferred_element_type=jnp.float32)
    b = jax.lax.dot_general(bo_ref[...].astype(jnp.bfloat16), ones_ref[...],
                            dimension_numbers=tn,
                            preferred_element_type=jnp.float32)
    return l + b


def _head_pass1_kernel(wo_ref, bo_ref, ones_ref, dT_ref, m_out_ref, s_out_ref,
                       m_ref, s_ref):
    j = pl.program_id(0)
    l = _logits_tile(wo_ref, bo_ref, ones_ref, dT_ref)
    row = jax.lax.broadcasted_iota(jnp.int32, (VT, 1), 0)
    l = jnp.where(row < VOCAB - j * VT, l, -1e30)
    m_tile = jnp.max(l, axis=0, keepdims=True)

    def _sumexp(x, m):
        return jnp.sum(jnp.exp(x - m), axis=0, keepdims=True)

    @pl.when(j == 0)
    def _first():
        m_ref[...] = m_tile
        s_ref[...] = _sumexp(l, m_tile)

    @pl.when(j > 0)
    def _rest():
        m_old = m_ref[...]
        m_new = jnp.maximum(m_old, m_tile)
        s_ref[...] = (s_ref[...] * jnp.exp(m_old - m_new)
                      + _sumexp(l, m_new))
        m_ref[...] = m_new

    @pl.when(j == NV - 1)
    def _emit():
        m_out_ref[...] = m_ref[...]
        s_out_ref[...] = 1.0 / s_ref[...]


def _head_pass2_kernel(wo_ref, bo_ref, ones_ref, dT_ref, m_ref, sinv_ref,
                       out_ref):
    l = _logits_tile(wo_ref, bo_ref, ones_ref, dT_ref)
    out_ref[...] = jnp.exp(l - m_ref[...]) * sinv_ref[...]


def _run_head(dT, Wo, bo_row):
    wt_spec = pl.BlockSpec((DENSE, VT), lambda j: (0, j))
    bo_spec = pl.BlockSpec((1, VT), lambda j: (0, j))
    dT_spec = pl.BlockSpec((DENSE, B), lambda j: (0, 0))
    row_spec = pl.BlockSpec((1, B), lambda j: (0, 0))
    ones_row = jnp.ones((1, B), jnp.bfloat16)

    m, sinv = pl.pallas_call(
        _head_pass1_kernel,
        grid=(NV,),
        in_specs=[wt_spec, bo_spec, row_spec, dT_spec],
        out_specs=[row_spec, row_spec],
        out_shape=[jax.ShapeDtypeStruct((1, B), jnp.float32),
                   jax.ShapeDtypeStruct((1, B), jnp.float32)],
        scratch_shapes=[pltpu.VMEM((1, B), jnp.float32),
                        pltpu.VMEM((1, B), jnp.float32)],
    )(Wo, bo_row, ones_row, dT)

    out_t = pl.pallas_call(
        _head_pass2_kernel,
        grid=(NV,),
        in_specs=[wt_spec, bo_spec, row_spec, dT_spec, row_spec, row_spec],
        out_specs=pl.BlockSpec((VT, B), lambda j: (j, 0)),
        out_shape=jax.ShapeDtypeStruct((VOCAB, B), jnp.float32),
    )(Wo, bo_row, ones_row, dT, m, sinv)
    return out_t


def kernel(inputs, training, emb_table, Wf_k, Wf_r, bf, Wb_k, Wb_r, bb,
           Wd, bd, Wo, bo):
    del training
    flat_idx = inputs.T.reshape(-1)
    x_tm = _sc_gather(emb_table, flat_idx).reshape(T, B, EMB)

    b16 = lambda w: w.astype(jnp.bfloat16)
    d = _run_lstm(x_tm, b16(Wf_k), b16(Wf_r), bf.reshape(1, -1),
                  b16(Wb_k), b16(Wb_r), bb.reshape(1, -1),
                  b16(Wd), bd.reshape(1, -1))
    dT = d.T

    out_t = _run_head(dT, Wo, bo.reshape(1, -1))
    return out_t.T
